# Initial kernel scaffold; baseline (speedup 1.0000x reference)
import jax
import jax.numpy as jnp
from jax.experimental import pallas as pl


def kernel(x, edge_index, W1l, W1r, b1, W2l, W2r, b2):
    raise NotImplementedError("write your pallas kernel here")


# speedup vs baseline: 3.3991x; 3.3991x over previous
import functools

import jax
import jax.numpy as jnp
from jax import lax
from jax.experimental import pallas as pl
from jax.experimental.pallas import tpu as pltpu
from jax.experimental.pallas import tpu_sc as plsc

N_NODES = 10000
N_EDGES = 160000
D_IN = 256
D_HID = 128
D_OUT = 64

NC = 2
NS = 16
NW = NC * NS

N_PAD = 10240
E_PAD = 163840
E_PER_TILE = E_PAD // NW
CHUNK = 128
N_ITERS = E_PER_TILE // CHUNK
ROWS_PER_TILE = N_PAD // NS
CNT_W = 16


def _make_sc_agg(D, with_cnt):
  mesh = plsc.VectorSubcoreMesh(core_axis_name="c", subcore_axis_name="s")
  out_type = [jax.ShapeDtypeStruct((NC, N_PAD, D), jnp.float32)]
  if with_cnt:
    out_type.append(jax.ShapeDtypeStruct((NC, N_PAD, CNT_W), jnp.float32))
  scratch = [
      pltpu.VMEM((CHUNK,), jnp.int32),
      pltpu.VMEM((CHUNK,), jnp.int32),
      pltpu.VMEM((CHUNK, D), jnp.float32),
      pltpu.VMEM_SHARED((N_PAD, D), jnp.float32),
      pltpu.SemaphoreType.DMA,
  ]
  if with_cnt:
    scratch += [
        pltpu.VMEM((CHUNK, CNT_W), jnp.float32),
        pltpu.VMEM_SHARED((N_PAD, CNT_W), jnp.float32),
    ]

  def body(p_hbm, src_hbm, dst_hbm, *rest):
    if with_cnt:
      (out_hbm, cnt_hbm, src_v, dst_v, rows_v, acc_sh, sem, ones_v,
       cnt_sh) = rest
    else:
      out_hbm, src_v, dst_v, rows_v, acc_sh, sem = rest
    cid = lax.axis_index("c")
    sid = lax.axis_index("s")
    wid = sid * NC + cid
    row0 = sid * ROWS_PER_TILE

    def zero_row(i, _):
      for j in range(D // 16):
        rows_v[i, pl.ds(j * 16, 16)] = jnp.zeros((16,), jnp.float32)
      return _
    lax.fori_loop(0, CHUNK, zero_row, 0)
    for r in range(ROWS_PER_TILE // CHUNK):
      pltpu.sync_copy(rows_v, acc_sh.at[pl.ds(row0 + r * CHUNK, CHUNK)])
    if with_cnt:
      def zero_ones(i, _):
        ones_v[i, :] = jnp.zeros((CNT_W,), jnp.float32)
        return _
      lax.fori_loop(0, CHUNK, zero_ones, 0)
      for r in range(ROWS_PER_TILE // CHUNK):
        pltpu.sync_copy(ones_v, cnt_sh.at[pl.ds(row0 + r * CHUNK, CHUNK)])
      def fill_ones(i, _):
        ones_v[i, :] = jnp.ones((CNT_W,), jnp.float32)
        return _
      lax.fori_loop(0, CHUNK, fill_ones, 0)
    plsc.subcore_barrier()

    e_base = wid * E_PER_TILE

    def step(k, _):
      base = e_base + k * CHUNK
      pltpu.sync_copy(src_hbm.at[pl.ds(base, CHUNK)], src_v)
      pltpu.sync_copy(dst_hbm.at[pl.ds(base, CHUNK)], dst_v)
      pltpu.async_copy(p_hbm.at[src_v], rows_v, sem).wait()
      pltpu.sync_copy(rows_v, acc_sh.at[dst_v], add=True)
      if with_cnt:
        pltpu.sync_copy(ones_v, cnt_sh.at[dst_v], add=True)
      return _
    lax.fori_loop(0, N_ITERS, step, 0)

    plsc.subcore_barrier()
    pltpu.sync_copy(acc_sh.at[pl.ds(row0, ROWS_PER_TILE)],
                    out_hbm.at[cid, pl.ds(row0, ROWS_PER_TILE)])
    if with_cnt:
      pltpu.sync_copy(cnt_sh.at[pl.ds(row0, ROWS_PER_TILE)],
                      cnt_hbm.at[cid, pl.ds(row0, ROWS_PER_TILE)])

  return pl.kernel(body, out_type=out_type, mesh=mesh, scratch_types=scratch,
                   compiler_params=pltpu.CompilerParams(
                       use_tc_tiling_on_sc=False))


_make_sc_agg = functools.lru_cache(maxsize=None)(_make_sc_agg)


def _agg_cnt_128(p, src, dst):
  return _make_sc_agg(D_HID, True)(p, src, dst)


def _agg_64(p, src, dst):
  return _make_sc_agg(D_OUT, False)(p, src, dst)


_BR = 1024


def _mm_body(x_ref, w_ref, o_ref):
  o_ref[...] = jnp.dot(x_ref[...], w_ref[...],
                       preferred_element_type=jnp.float32)


def _matmul(x, w):
  m, k = x.shape
  _, n = w.shape
  return pl.pallas_call(
      _mm_body,
      grid=(m // _BR,),
      in_specs=[
          pl.BlockSpec((_BR, k), lambda i: (i, 0)),
          pl.BlockSpec((k, n), lambda i: (0, 0)),
      ],
      out_specs=pl.BlockSpec((_BR, n), lambda i: (i, 0)),
      out_shape=jax.ShapeDtypeStruct((m, n), jnp.float32),
  )(x, w)


def _mid_body(a0_ref, a1_ref, c0_ref, c1_ref, xr_ref, b_ref, w_ref, o_ref):
  cnt = c0_ref[:, 0:1] + c1_ref[:, 0:1]
  rcnt = 1.0 / jnp.maximum(cnt, 1.0)
  h = (a0_ref[...] + a1_ref[...]) * rcnt + xr_ref[...] + b_ref[...]
  h = jnp.maximum(h, 0.0)
  o_ref[...] = jnp.dot(h, w_ref[...], preferred_element_type=jnp.float32)


def _layer_mid(a0, a1, c0, c1, xr, b, w):
  m, d = a0.shape
  _, n = w.shape
  return pl.pallas_call(
      _mid_body,
      grid=(m // _BR,),
      in_specs=[
          pl.BlockSpec((_BR, d), lambda i: (i, 0)),
          pl.BlockSpec((_BR, d), lambda i: (i, 0)),
          pl.BlockSpec((_BR, CNT_W), lambda i: (i, 0)),
          pl.BlockSpec((_BR, CNT_W), lambda i: (i, 0)),
          pl.BlockSpec((_BR, d), lambda i: (i, 0)),
          pl.BlockSpec((1, d), lambda i: (0, 0)),
          pl.BlockSpec((d, n), lambda i: (0, 0)),
      ],
      out_specs=pl.BlockSpec((_BR, n), lambda i: (i, 0)),
      out_shape=jax.ShapeDtypeStruct((m, n), jnp.float32),
  )(a0, a1, c0, c1, xr, b, w)


def _out_body(a0_ref, a1_ref, c0_ref, c1_ref, hr_ref, b_ref, o_ref):
  cnt = c0_ref[:, 0:1] + c1_ref[:, 0:1]
  rcnt = 1.0 / jnp.maximum(cnt, 1.0)
  o = (a0_ref[...] + a1_ref[...]) * rcnt + hr_ref[...] + b_ref[...]
  m = jnp.max(o, axis=-1, keepdims=True)
  e = jnp.exp(o - m)
  lse = m + jnp.log(jnp.sum(e, axis=-1, keepdims=True))
  o_ref[...] = o - lse


def _layer_out(a0, a1, c0, c1, hr, b):
  m, d = a0.shape
  return pl.pallas_call(
      _out_body,
      grid=(m // _BR,),
      in_specs=[
          pl.BlockSpec((_BR, d), lambda i: (i, 0)),
          pl.BlockSpec((_BR, d), lambda i: (i, 0)),
          pl.BlockSpec((_BR, CNT_W), lambda i: (i, 0)),
          pl.BlockSpec((_BR, CNT_W), lambda i: (i, 0)),
          pl.BlockSpec((_BR, d), lambda i: (i, 0)),
          pl.BlockSpec((1, d), lambda i: (0, 0)),
      ],
      out_specs=pl.BlockSpec((_BR, d), lambda i: (i, 0)),
      out_shape=jax.ShapeDtypeStruct((m, d), jnp.float32),
  )(a0, a1, c0, c1, hr, b)


def kernel(x, edge_index, W1l, W1r, b1, W2l, W2r, b2):
  ei = edge_index.astype(jnp.int32)
  pad = jnp.full((E_PAD - N_EDGES,), N_PAD - 1, jnp.int32)
  src = jnp.concatenate([ei[0], pad])
  dst = jnp.concatenate([ei[1], pad])
  x_pad = jnp.pad(x, ((0, N_PAD - N_NODES), (0, 0)))

  wcat1 = jnp.concatenate([W1l.T, W1r.T], axis=1)
  pcat = _matmul(x_pad, wcat1)
  p1 = pcat[:, :D_HID]
  xr = pcat[:, D_HID:]

  agg1, cnt = _agg_cnt_128(p1, src, dst)

  wcat2 = jnp.concatenate([W2l.T, W2r.T], axis=1)
  out2 = _layer_mid(agg1[0], agg1[1], cnt[0], cnt[1], xr,
                    b1.reshape(1, -1), wcat2)
  p2 = out2[:, :D_OUT]
  hr = out2[:, D_OUT:]

  agg2 = _agg_64(p2, src, dst)
  if isinstance(agg2, (list, tuple)):
    agg2 = agg2[0]

  out = _layer_out(agg2[0], agg2[1], cnt[0], cnt[1], hr, b2.reshape(1, -1))
  return out[:N_NODES]


# speedup vs baseline: 5.2378x; 1.5409x over previous
import functools

import jax
import jax.numpy as jnp
from jax import lax
from jax.experimental import pallas as pl
from jax.experimental.pallas import tpu as pltpu
from jax.experimental.pallas import tpu_sc as plsc

N_NODES = 10000
N_EDGES = 160000
D_IN = 256
D_HID = 128
D_OUT = 64

NC = 2
NS = 16
NW = NC * NS

N_PAD = 10240
E_PAD = 163840
E_PER_TILE = E_PAD // NW
CHUNK = 128
N_ITERS = E_PER_TILE // CHUNK
ROWS_PER_TILE = N_PAD // NS
CNT_W = 8


NBUF = 4
D_HALF = 64


def _make_sc_agg(n_chunks, feature_split, with_cnt):
  D = D_HALF
  mesh = plsc.VectorSubcoreMesh(core_axis_name="c", subcore_axis_name="s")
  table_rows = 2 * N_PAD if feature_split else N_PAD
  out_type = [jax.ShapeDtypeStruct((NC, N_PAD, D), jnp.float32)]
  if with_cnt:
    out_type.append(jax.ShapeDtypeStruct((NC, N_PAD, CNT_W), jnp.float32))
  scratch = [
      pltpu.VMEM((n_chunks, CHUNK), jnp.int32),
      pltpu.VMEM((n_chunks, CHUNK), jnp.int32),
      [pltpu.VMEM((CHUNK, D), jnp.float32) for _ in range(NBUF)],
      pltpu.VMEM_SHARED((N_PAD, D), jnp.float32),
      [pltpu.SemaphoreType.DMA for _ in range(NBUF)],
      [pltpu.SemaphoreType.DMA for _ in range(NBUF)],
  ]
  if with_cnt:
    scratch += [
        pltpu.VMEM((CHUNK, CNT_W), jnp.float32),
        pltpu.VMEM_SHARED((N_PAD, CNT_W), jnp.float32),
    ]

  def body(p_hbm, src_hbm, dst_hbm, *rest):
    if with_cnt:
      (out_hbm, cnt_hbm, src_v, dst_v, rows, acc_sh, sem_g, sem_s, ones_v,
       cnt_sh) = rest
    else:
      out_hbm, src_v, dst_v, rows, acc_sh, sem_g, sem_s = rest
    cid = lax.axis_index("c")
    sid = lax.axis_index("s")
    row0 = sid * ROWS_PER_TILE
    if feature_split:
      chunk0 = sid * n_chunks
    else:
      chunk0 = (sid * NC + cid) * n_chunks

    pltpu.sync_copy(src_hbm.at[pl.ds(chunk0, n_chunks)], src_v)
    pltpu.sync_copy(dst_hbm.at[pl.ds(chunk0, n_chunks)], dst_v)
    if feature_split:
      off = (cid * N_PAD).astype(jnp.int32)

      def adjust(i, carry):
        for j in range(CHUNK // 16):
          sl = src_v[i, pl.ds(j * 16, 16)]
          src_v[i, pl.ds(j * 16, 16)] = sl + off
        return carry
      lax.fori_loop(0, n_chunks, adjust, 0)

    def zero_row(i, carry):
      for j in range(D // 16):
        rows[0][i, pl.ds(j * 16, 16)] = jnp.zeros((16,), jnp.float32)
      return carry
    lax.fori_loop(0, CHUNK, zero_row, 0)
    for r in range(ROWS_PER_TILE // CHUNK):
      pltpu.sync_copy(rows[0], acc_sh.at[pl.ds(row0 + r * CHUNK, CHUNK)])
    if with_cnt:
      def zero_ones(i, carry):
        ones_v[i, :] = jnp.zeros((CNT_W,), jnp.float32)
        return carry
      lax.fori_loop(0, CHUNK, zero_ones, 0)
      for r in range(ROWS_PER_TILE // CHUNK):
        pltpu.sync_copy(ones_v, cnt_sh.at[pl.ds(row0 + r * CHUNK, CHUNK)])
      def fill_ones(i, carry):
        ones_v[i, :] = jnp.ones((CNT_W,), jnp.float32)
        return carry
      lax.fori_loop(0, CHUNK, fill_ones, 0)
    plsc.subcore_barrier()

    def gather(k, b):
      pltpu.async_copy(p_hbm.at[src_v.at[k]], rows[b], sem_g[b])

    def scatter(k, b):
      pltpu.async_copy(rows[b], acc_sh.at[dst_v.at[k]], sem_s[b], add=True)

    for b in range(NBUF - 1):
      gather(b, b)

    def step(j, carry):
      for b in range(NBUF):
        k = j * NBUF + b
        pltpu.make_async_copy(p_hbm.at[src_v.at[k]], rows[b],
                              sem_g[b]).wait()
        scatter(k, b)
        if with_cnt:
          pltpu.sync_copy(ones_v, cnt_sh.at[dst_v.at[k]], add=True)
        bn = (b + NBUF - 1) % NBUF
        kn = k + NBUF - 1

        @pl.when(k >= 1)
        def _wait_prev(bn=bn, kn=kn):
          pltpu.make_async_copy(rows[bn], acc_sh.at[dst_v.at[kn - NBUF]],
                                sem_s[bn]).wait()

        @pl.when(kn < n_chunks)
        def _prefetch(bn=bn, kn=kn):
          gather(kn, bn)
      return carry
    lax.fori_loop(0, n_chunks // NBUF, step, 0)
    bl = (n_chunks - 1) % NBUF
    pltpu.make_async_copy(rows[bl], acc_sh.at[dst_v.at[n_chunks - 1]],
                          sem_s[bl]).wait()

    plsc.subcore_barrier()
    pltpu.sync_copy(acc_sh.at[pl.ds(row0, ROWS_PER_TILE)],
                    out_hbm.at[cid, pl.ds(row0, ROWS_PER_TILE)])
    if with_cnt:
      pltpu.sync_copy(cnt_sh.at[pl.ds(row0, ROWS_PER_TILE)],
                      cnt_hbm.at[cid, pl.ds(row0, ROWS_PER_TILE)])

  return pl.kernel(body, out_type=out_type, mesh=mesh, scratch_types=scratch,
                   compiler_params=pltpu.CompilerParams(
                       use_tc_tiling_on_sc=False))


_make_sc_agg = functools.lru_cache(maxsize=None)(_make_sc_agg)


def _agg_l1(p_split, src, dst):
  return _make_sc_agg(E_PAD // CHUNK // NS, True, True)(p_split, src, dst)


def _agg_l2(p, src, dst):
  out = _make_sc_agg(E_PAD // CHUNK // NW, False, False)(p, src, dst)
  if isinstance(out, (list, tuple)):
    out = out[0]
  return out


_BR = 1024


def _mm_body(x_ref, w_ref, o_ref):
  o_ref[...] = jnp.dot(x_ref[...], w_ref[...],
                       preferred_element_type=jnp.float32)


def _matmul(x, w):
  m, k = x.shape
  _, n = w.shape
  return pl.pallas_call(
      _mm_body,
      grid=(m // _BR,),
      in_specs=[
          pl.BlockSpec((_BR, k), lambda i: (i, 0)),
          pl.BlockSpec((k, n), lambda i: (0, 0)),
      ],
      out_specs=pl.BlockSpec((_BR, n), lambda i: (i, 0)),
      out_shape=jax.ShapeDtypeStruct((m, n), jnp.float32),
  )(x, w)


def _mid_body(a_lo_ref, a_hi_ref, c_ref, xr_ref, b_ref, w_ref, o_ref):
  rcnt = 1.0 / jnp.maximum(c_ref[:, 0:1], 1.0)
  s = jnp.concatenate([a_lo_ref[...], a_hi_ref[...]], axis=1)
  h = s * rcnt + xr_ref[...] + b_ref[...]
  h = jnp.maximum(h, 0.0)
  o_ref[...] = jnp.dot(h, w_ref[...], preferred_element_type=jnp.float32)


def _layer_mid(a_lo, a_hi, c, xr, b, w):
  m, d = a_lo.shape
  _, n = w.shape
  return pl.pallas_call(
      _mid_body,
      grid=(m // _BR,),
      in_specs=[
          pl.BlockSpec((_BR, d), lambda i: (i, 0)),
          pl.BlockSpec((_BR, d), lambda i: (i, 0)),
          pl.BlockSpec((_BR, CNT_W), lambda i: (i, 0)),
          pl.BlockSpec((_BR, 2 * d), lambda i: (i, 0)),
          pl.BlockSpec((1, 2 * d), lambda i: (0, 0)),
          pl.BlockSpec((2 * d, n), lambda i: (0, 0)),
      ],
      out_specs=pl.BlockSpec((_BR, n), lambda i: (i, 0)),
      out_shape=jax.ShapeDtypeStruct((m, n), jnp.float32),
  )(a_lo, a_hi, c, xr, b, w)


def _out_body(a0_ref, a1_ref, c_ref, hr_ref, b_ref, o_ref):
  rcnt = 1.0 / jnp.maximum(c_ref[:, 0:1], 1.0)
  o = (a0_ref[...] + a1_ref[...]) * rcnt + hr_ref[...] + b_ref[...]
  m = jnp.max(o, axis=-1, keepdims=True)
  e = jnp.exp(o - m)
  lse = m + jnp.log(jnp.sum(e, axis=-1, keepdims=True))
  o_ref[...] = o - lse


def _layer_out(a0, a1, c, hr, b):
  m, d = a0.shape
  return pl.pallas_call(
      _out_body,
      grid=(m // _BR,),
      in_specs=[
          pl.BlockSpec((_BR, d), lambda i: (i, 0)),
          pl.BlockSpec((_BR, d), lambda i: (i, 0)),
          pl.BlockSpec((_BR, CNT_W), lambda i: (i, 0)),
          pl.BlockSpec((_BR, d), lambda i: (i, 0)),
          pl.BlockSpec((1, d), lambda i: (0, 0)),
      ],
      out_specs=pl.BlockSpec((_BR, d), lambda i: (i, 0)),
      out_shape=jax.ShapeDtypeStruct((m, d), jnp.float32),
  )(a0, a1, c, hr, b)


def kernel(x, edge_index, W1l, W1r, b1, W2l, W2r, b2):
  ei = edge_index.astype(jnp.int32)
  pad = jnp.full((E_PAD - N_EDGES,), N_PAD - 1, jnp.int32)
  src = jnp.concatenate([ei[0], pad]).reshape(E_PAD // CHUNK, CHUNK)
  dst = jnp.concatenate([ei[1], pad]).reshape(E_PAD // CHUNK, CHUNK)
  x_pad = jnp.pad(x, ((0, N_PAD - N_NODES), (0, 0)))

  wcat1 = jnp.concatenate([W1l.T, W1r.T], axis=1)
  pcat = _matmul(x_pad, wcat1)
  p1 = pcat[:, :D_HID]
  xr = pcat[:, D_HID:]
  p_split = jnp.concatenate([p1[:, :D_HALF], p1[:, D_HALF:]], axis=0)

  agg1, cnt = _agg_l1(p_split, src, dst)

  wcat2 = jnp.concatenate([W2l.T, W2r.T], axis=1)
  out2 = _layer_mid(agg1[0], agg1[1], cnt[0], xr, b1.reshape(1, -1), wcat2)
  p2 = out2[:, :D_OUT]
  hr = out2[:, D_OUT:]

  agg2 = _agg_l2(p2, src, dst)

  out = _layer_out(agg2[0], agg2[1], cnt[0], hr, b2.reshape(1, -1))
  return out[:N_NODES]


# speedup vs baseline: 5.2408x; 1.0006x over previous
import functools

import jax
import jax.numpy as jnp
from jax import lax
from jax.experimental import pallas as pl
from jax.experimental.pallas import tpu as pltpu
from jax.experimental.pallas import tpu_sc as plsc

N_NODES = 10000
N_EDGES = 160000
D_IN = 256
D_HID = 128
D_OUT = 64

NC = 2
NS = 16
NW = NC * NS

N_PAD = 10240
E_PAD = 163840
E_PER_TILE = E_PAD // NW
CHUNK = 128
N_ITERS = E_PER_TILE // CHUNK
ROWS_PER_TILE = N_PAD // NS
CNT_W = 8


D_HALF = 64


def _make_sc_agg(n_chunks, feature_split, with_cnt, NBUF):
  D = D_HALF
  mesh = plsc.VectorSubcoreMesh(core_axis_name="c", subcore_axis_name="s")
  table_rows = 2 * N_PAD if feature_split else N_PAD
  out_type = [jax.ShapeDtypeStruct((NC, N_PAD, D), jnp.float32)]
  if with_cnt:
    out_type.append(jax.ShapeDtypeStruct((NC, N_PAD, CNT_W), jnp.float32))
  scratch = [
      pltpu.VMEM((n_chunks, CHUNK), jnp.int32),
      pltpu.VMEM((n_chunks, CHUNK), jnp.int32),
      [pltpu.VMEM((CHUNK, D), jnp.float32) for _ in range(NBUF)],
      pltpu.VMEM_SHARED((N_PAD, D), jnp.float32),
      [pltpu.SemaphoreType.DMA for _ in range(NBUF)],
      [pltpu.SemaphoreType.DMA for _ in range(NBUF)],
  ]
  if with_cnt:
    scratch += [
        pltpu.VMEM((CHUNK, CNT_W), jnp.float32),
        pltpu.VMEM_SHARED((N_PAD, CNT_W), jnp.float32),
    ]

  def body(p_hbm, src_hbm, dst_hbm, *rest):
    if with_cnt:
      (out_hbm, cnt_hbm, src_v, dst_v, rows, acc_sh, sem_g, sem_s, ones_v,
       cnt_sh) = rest
    else:
      out_hbm, src_v, dst_v, rows, acc_sh, sem_g, sem_s = rest
    cid = lax.axis_index("c")
    sid = lax.axis_index("s")
    row0 = sid * ROWS_PER_TILE
    if feature_split:
      chunk0 = sid * n_chunks
    else:
      chunk0 = (sid * NC + cid) * n_chunks

    pltpu.sync_copy(src_hbm.at[pl.ds(chunk0, n_chunks)], src_v)
    pltpu.sync_copy(dst_hbm.at[pl.ds(chunk0, n_chunks)], dst_v)
    if feature_split:
      off = (cid * N_PAD).astype(jnp.int32)

      def adjust(i, carry):
        for j in range(CHUNK // 16):
          sl = src_v[i, pl.ds(j * 16, 16)]
          src_v[i, pl.ds(j * 16, 16)] = sl + off
        return carry
      lax.fori_loop(0, n_chunks, adjust, 0)

    def zero_row(i, carry):
      for j in range(D // 16):
        rows[0][i, pl.ds(j * 16, 16)] = jnp.zeros((16,), jnp.float32)
      return carry
    lax.fori_loop(0, CHUNK, zero_row, 0)
    for r in range(ROWS_PER_TILE // CHUNK):
      pltpu.sync_copy(rows[0], acc_sh.at[pl.ds(row0 + r * CHUNK, CHUNK)])
    if with_cnt:
      def zero_ones(i, carry):
        ones_v[i, :] = jnp.zeros((CNT_W,), jnp.float32)
        return carry
      lax.fori_loop(0, CHUNK, zero_ones, 0)
      for r in range(ROWS_PER_TILE // CHUNK):
        pltpu.sync_copy(ones_v, cnt_sh.at[pl.ds(row0 + r * CHUNK, CHUNK)])
      def fill_ones(i, carry):
        ones_v[i, :] = jnp.ones((CNT_W,), jnp.float32)
        return carry
      lax.fori_loop(0, CHUNK, fill_ones, 0)
    plsc.subcore_barrier()

    def gather(k, b):
      pltpu.async_copy(p_hbm.at[src_v.at[k]], rows[b], sem_g[b])

    def scatter(k, b):
      pltpu.async_copy(rows[b], acc_sh.at[dst_v.at[k]], sem_s[b], add=True)

    for b in range(NBUF - 1):
      gather(b, b)

    def step(j, carry):
      for b in range(NBUF):
        k = j * NBUF + b
        pltpu.make_async_copy(p_hbm.at[src_v.at[k]], rows[b],
                              sem_g[b]).wait()
        scatter(k, b)
        if with_cnt:
          pltpu.sync_copy(ones_v, cnt_sh.at[dst_v.at[k]], add=True)
        bn = (b + NBUF - 1) % NBUF
        kn = k + NBUF - 1

        @pl.when(k >= 1)
        def _wait_prev(bn=bn, kn=kn):
          pltpu.make_async_copy(rows[bn], acc_sh.at[dst_v.at[kn - NBUF]],
                                sem_s[bn]).wait()

        @pl.when(kn < n_chunks)
        def _prefetch(bn=bn, kn=kn):
          gather(kn, bn)
      return carry
    lax.fori_loop(0, n_chunks // NBUF, step, 0)
    bl = (n_chunks - 1) % NBUF
    pltpu.make_async_copy(rows[bl], acc_sh.at[dst_v.at[n_chunks - 1]],
                          sem_s[bl]).wait()

    plsc.subcore_barrier()
    pltpu.sync_copy(acc_sh.at[pl.ds(row0, ROWS_PER_TILE)],
                    out_hbm.at[cid, pl.ds(row0, ROWS_PER_TILE)])
    if with_cnt:
      pltpu.sync_copy(cnt_sh.at[pl.ds(row0, ROWS_PER_TILE)],
                      cnt_hbm.at[cid, pl.ds(row0, ROWS_PER_TILE)])

  return pl.kernel(body, out_type=out_type, mesh=mesh, scratch_types=scratch,
                   compiler_params=pltpu.CompilerParams(
                       use_tc_tiling_on_sc=False))


_make_sc_agg = functools.lru_cache(maxsize=None)(_make_sc_agg)


def _agg_l1(p_split, src, dst):
  return _make_sc_agg(E_PAD // CHUNK // NS, True, True, 5)(p_split, src, dst)


def _agg_l2(p, src, dst):
  out = _make_sc_agg(E_PAD // CHUNK // NW, False, False, 8)(p, src, dst)
  if isinstance(out, (list, tuple)):
    out = out[0]
  return out


_BR = 1024


def _mm_body(x_ref, w_ref, o_ref):
  o_ref[...] = jnp.dot(x_ref[...], w_ref[...],
                       preferred_element_type=jnp.float32)


def _matmul(x, w):
  m, k = x.shape
  _, n = w.shape
  return pl.pallas_call(
      _mm_body,
      grid=(m // _BR,),
      in_specs=[
          pl.BlockSpec((_BR, k), lambda i: (i, 0)),
          pl.BlockSpec((k, n), lambda i: (0, 0)),
      ],
      out_specs=pl.BlockSpec((_BR, n), lambda i: (i, 0)),
      out_shape=jax.ShapeDtypeStruct((m, n), jnp.float32),
  )(x, w)


def _mid_body(a_lo_ref, a_hi_ref, c_ref, xr_ref, b_ref, w_ref, o_ref):
  rcnt = 1.0 / jnp.maximum(c_ref[:, 0:1], 1.0)
  s = jnp.concatenate([a_lo_ref[...], a_hi_ref[...]], axis=1)
  h = s * rcnt + xr_ref[...] + b_ref[...]
  h = jnp.maximum(h, 0.0)
  o_ref[...] = jnp.dot(h, w_ref[...], preferred_element_type=jnp.float32)


def _layer_mid(a_lo, a_hi, c, xr, b, w):
  m, d = a_lo.shape
  _, n = w.shape
  return pl.pallas_call(
      _mid_body,
      grid=(m // _BR,),
      in_specs=[
          pl.BlockSpec((_BR, d), lambda i: (i, 0)),
          pl.BlockSpec((_BR, d), lambda i: (i, 0)),
          pl.BlockSpec((_BR, CNT_W), lambda i: (i, 0)),
          pl.BlockSpec((_BR, 2 * d), lambda i: (i, 0)),
          pl.BlockSpec((1, 2 * d), lambda i: (0, 0)),
          pl.BlockSpec((2 * d, n), lambda i: (0, 0)),
      ],
      out_specs=pl.BlockSpec((_BR, n), lambda i: (i, 0)),
      out_shape=jax.ShapeDtypeStruct((m, n), jnp.float32),
  )(a_lo, a_hi, c, xr, b, w)


def _out_body(a0_ref, a1_ref, c_ref, hr_ref, b_ref, o_ref):
  rcnt = 1.0 / jnp.maximum(c_ref[:, 0:1], 1.0)
  o = (a0_ref[...] + a1_ref[...]) * rcnt + hr_ref[...] + b_ref[...]
  m = jnp.max(o, axis=-1, keepdims=True)
  e = jnp.exp(o - m)
  lse = m + jnp.log(jnp.sum(e, axis=-1, keepdims=True))
  o_ref[...] = o - lse


def _layer_out(a0, a1, c, hr, b):
  m, d = a0.shape
  return pl.pallas_call(
      _out_body,
      grid=(m // _BR,),
      in_specs=[
          pl.BlockSpec((_BR, d), lambda i: (i, 0)),
          pl.BlockSpec((_BR, d), lambda i: (i, 0)),
          pl.BlockSpec((_BR, CNT_W), lambda i: (i, 0)),
          pl.BlockSpec((_BR, d), lambda i: (i, 0)),
          pl.BlockSpec((1, d), lambda i: (0, 0)),
      ],
      out_specs=pl.BlockSpec((_BR, d), lambda i: (i, 0)),
      out_shape=jax.ShapeDtypeStruct((m, d), jnp.float32),
  )(a0, a1, c, hr, b)


def kernel(x, edge_index, W1l, W1r, b1, W2l, W2r, b2):
  ei = edge_index.astype(jnp.int32)
  pad = jnp.full((E_PAD - N_EDGES,), N_PAD - 1, jnp.int32)
  src = jnp.concatenate([ei[0], pad]).reshape(E_PAD // CHUNK, CHUNK)
  dst = jnp.concatenate([ei[1], pad]).reshape(E_PAD // CHUNK, CHUNK)
  x_pad = jnp.pad(x, ((0, N_PAD - N_NODES), (0, 0)))

  wcat1 = jnp.concatenate([W1l.T, W1r.T], axis=1)
  pcat = _matmul(x_pad, wcat1)
  p1 = pcat[:, :D_HID]
  xr = pcat[:, D_HID:]
  p_split = jnp.concatenate([p1[:, :D_HALF], p1[:, D_HALF:]], axis=0)

  agg1, cnt = _agg_l1(p_split, src, dst)

  wcat2 = jnp.concatenate([W2l.T, W2r.T], axis=1)
  out2 = _layer_mid(agg1[0], agg1[1], cnt[0], xr, b1.reshape(1, -1), wcat2)
  p2 = out2[:, :D_OUT]
  hr = out2[:, D_OUT:]

  agg2 = _agg_l2(p2, src, dst)

  out = _layer_out(agg2[0], agg2[1], cnt[0], hr, b2.reshape(1, -1))
  return out[:N_NODES]


# speedup vs baseline: 9.5056x; 1.8137x over previous
import functools

import jax
import jax.numpy as jnp
from jax import lax
from jax.experimental import pallas as pl
from jax.experimental.pallas import tpu as pltpu
from jax.experimental.pallas import tpu_sc as plsc

N_NODES = 10000
N_EDGES = 160000
D_IN = 256
D_HID = 128
D_OUT = 64

NC = 2
NS = 16
NW = NC * NS

N_PAD = 10240
E_PAD = 163840
E_PER_TILE = E_PAD // NW
CHUNK = 128
N_ITERS = E_PER_TILE // CHUNK
ROWS_PER_TILE = N_PAD // NS
CNT_W = 8


D_HALF = 64


def _make_sc_agg(n_chunks, feature_split, with_cnt, NBUF):
  D = D_HALF
  mesh = plsc.VectorSubcoreMesh(core_axis_name="c", subcore_axis_name="s")
  table_rows = 2 * N_PAD if feature_split else N_PAD
  out_type = [jax.ShapeDtypeStruct((NC, N_PAD, D), jnp.float32)]
  if with_cnt:
    out_type.append(jax.ShapeDtypeStruct((NC, N_PAD, CNT_W), jnp.float32))
  scratch = [
      pltpu.VMEM((n_chunks, CHUNK), jnp.int32),
      pltpu.VMEM((n_chunks, CHUNK), jnp.int32),
      [pltpu.VMEM((CHUNK, D), jnp.float32) for _ in range(NBUF)],
      pltpu.VMEM_SHARED((N_PAD, D), jnp.float32),
      [pltpu.SemaphoreType.DMA for _ in range(NBUF)],
      [pltpu.SemaphoreType.DMA for _ in range(NBUF)],
  ]
  if with_cnt:
    scratch += [
        pltpu.VMEM((CHUNK, CNT_W), jnp.float32),
        pltpu.VMEM_SHARED((N_PAD, CNT_W), jnp.float32),
    ]

  def body(p_hbm, src_hbm, dst_hbm, *rest):
    if with_cnt:
      (out_hbm, cnt_hbm, src_v, dst_v, rows, acc_sh, sem_g, sem_s, ones_v,
       cnt_sh) = rest
    else:
      out_hbm, src_v, dst_v, rows, acc_sh, sem_g, sem_s = rest
    cid = lax.axis_index("c")
    sid = lax.axis_index("s")
    row0 = sid * ROWS_PER_TILE
    if feature_split:
      chunk0 = sid * n_chunks
    else:
      chunk0 = (sid * NC + cid) * n_chunks

    pltpu.sync_copy(src_hbm.at[pl.ds(chunk0, n_chunks)], src_v)
    pltpu.sync_copy(dst_hbm.at[pl.ds(chunk0, n_chunks)], dst_v)
    if feature_split:
      off = (cid * N_PAD).astype(jnp.int32)

      def adjust(i, carry):
        for j in range(CHUNK // 16):
          sl = src_v[i, pl.ds(j * 16, 16)]
          src_v[i, pl.ds(j * 16, 16)] = sl + off
        return carry
      lax.fori_loop(0, n_chunks, adjust, 0)

    def zero_row(i, carry):
      for j in range(D // 16):
        rows[0][i, pl.ds(j * 16, 16)] = jnp.zeros((16,), jnp.float32)
      return carry
    lax.fori_loop(0, CHUNK, zero_row, 0)
    for r in range(ROWS_PER_TILE // CHUNK):
      pltpu.sync_copy(rows[0], acc_sh.at[pl.ds(row0 + r * CHUNK, CHUNK)])
    if with_cnt:
      def zero_ones(i, carry):
        ones_v[i, :] = jnp.zeros((CNT_W,), jnp.float32)
        return carry
      lax.fori_loop(0, CHUNK, zero_ones, 0)
      for r in range(ROWS_PER_TILE // CHUNK):
        pltpu.sync_copy(ones_v, cnt_sh.at[pl.ds(row0 + r * CHUNK, CHUNK)])
      def fill_ones(i, carry):
        ones_v[i, :] = jnp.ones((CNT_W,), jnp.float32)
        return carry
      lax.fori_loop(0, CHUNK, fill_ones, 0)
    plsc.subcore_barrier()

    def gather(k, b):
      pltpu.async_copy(p_hbm.at[src_v.at[k]], rows[b], sem_g[b])

    def scatter(k, b):
      pltpu.async_copy(rows[b], acc_sh.at[dst_v.at[k]], sem_s[b], add=True)

    for b in range(NBUF - 1):
      gather(b, b)

    def step(j, carry):
      for b in range(NBUF):
        k = j * NBUF + b
        pltpu.make_async_copy(p_hbm.at[src_v.at[k]], rows[b],
                              sem_g[b]).wait()
        scatter(k, b)
        if with_cnt:
          pltpu.sync_copy(ones_v, cnt_sh.at[dst_v.at[k]], add=True)
        bn = (b + NBUF - 1) % NBUF
        kn = k + NBUF - 1

        @pl.when(k >= 1)
        def _wait_prev(bn=bn, kn=kn):
          pltpu.make_async_copy(rows[bn], acc_sh.at[dst_v.at[kn - NBUF]],
                                sem_s[bn]).wait()

        @pl.when(kn < n_chunks)
        def _prefetch(bn=bn, kn=kn):
          gather(kn, bn)
      return carry
    lax.fori_loop(0, n_chunks // NBUF, step, 0)
    bl = (n_chunks - 1) % NBUF
    pltpu.make_async_copy(rows[bl], acc_sh.at[dst_v.at[n_chunks - 1]],
                          sem_s[bl]).wait()

    plsc.subcore_barrier()
    pltpu.sync_copy(acc_sh.at[pl.ds(row0, ROWS_PER_TILE)],
                    out_hbm.at[cid, pl.ds(row0, ROWS_PER_TILE)])
    if with_cnt:
      pltpu.sync_copy(cnt_sh.at[pl.ds(row0, ROWS_PER_TILE)],
                      cnt_hbm.at[cid, pl.ds(row0, ROWS_PER_TILE)])

  return pl.kernel(body, out_type=out_type, mesh=mesh, scratch_types=scratch,
                   compiler_params=pltpu.CompilerParams(
                       use_tc_tiling_on_sc=False))


_make_sc_agg = functools.lru_cache(maxsize=None)(_make_sc_agg)


def _agg_l1(p_split, src, dst):
  return _make_sc_agg(E_PAD // CHUNK // NS, True, True, 5)(p_split, src, dst)


def _agg_l2(p, src, dst):
  out = _make_sc_agg(E_PAD // CHUNK // NW, False, False, 8)(p, src, dst)
  if isinstance(out, (list, tuple)):
    out = out[0]
  return out


_BR = 1024


def _mm_body(x_ref, w_ref, o_ref):
  o_ref[...] = jnp.dot(x_ref[...], w_ref[...],
                       preferred_element_type=jnp.float32)


def _matmul(x, w):
  m, k = x.shape
  _, n = w.shape
  return pl.pallas_call(
      _mm_body,
      grid=(m // _BR,),
      in_specs=[
          pl.BlockSpec((_BR, k), lambda i: (i, 0)),
          pl.BlockSpec((k, n), lambda i: (0, 0)),
      ],
      out_specs=pl.BlockSpec((_BR, n), lambda i: (i, 0)),
      out_shape=jax.ShapeDtypeStruct((m, n), jnp.float32),
  )(x, w)


def _mid_body(a_lo_ref, a_hi_ref, c_ref, xr_ref, b_ref, w_ref, o_ref):
  rcnt = 1.0 / jnp.maximum(c_ref[:, 0:1], 1.0)
  s = jnp.concatenate([a_lo_ref[...], a_hi_ref[...]], axis=1)
  h = s * rcnt + xr_ref[...] + b_ref[...]
  h = jnp.maximum(h, 0.0)
  o_ref[...] = jnp.dot(h, w_ref[...], preferred_element_type=jnp.float32)


def _layer_mid(a_lo, a_hi, c, xr, b, w):
  m, d = a_lo.shape
  _, n = w.shape
  return pl.pallas_call(
      _mid_body,
      grid=(m // _BR,),
      in_specs=[
          pl.BlockSpec((_BR, d), lambda i: (i, 0)),
          pl.BlockSpec((_BR, d), lambda i: (i, 0)),
          pl.BlockSpec((_BR, CNT_W), lambda i: (i, 0)),
          pl.BlockSpec((_BR, 2 * d), lambda i: (i, 0)),
          pl.BlockSpec((1, 2 * d), lambda i: (0, 0)),
          pl.BlockSpec((2 * d, n), lambda i: (0, 0)),
      ],
      out_specs=pl.BlockSpec((_BR, n), lambda i: (i, 0)),
      out_shape=jax.ShapeDtypeStruct((m, n), jnp.float32),
  )(a_lo, a_hi, c, xr, b, w)


def _out_body(a0_ref, a1_ref, c_ref, hr_ref, b_ref, o_ref):
  rcnt = 1.0 / jnp.maximum(c_ref[:, 0:1], 1.0)
  o = (a0_ref[...] + a1_ref[...]) * rcnt + hr_ref[...] + b_ref[...]
  m = jnp.max(o, axis=-1, keepdims=True)
  e = jnp.exp(o - m)
  lse = m + jnp.log(jnp.sum(e, axis=-1, keepdims=True))
  o_ref[...] = o - lse


def _layer_out(a0, a1, c, hr, b):
  m, d = a0.shape
  return pl.pallas_call(
      _out_body,
      grid=(m // _BR,),
      in_specs=[
          pl.BlockSpec((_BR, d), lambda i: (i, 0)),
          pl.BlockSpec((_BR, d), lambda i: (i, 0)),
          pl.BlockSpec((_BR, CNT_W), lambda i: (i, 0)),
          pl.BlockSpec((_BR, d), lambda i: (i, 0)),
          pl.BlockSpec((1, d), lambda i: (0, 0)),
      ],
      out_specs=pl.BlockSpec((_BR, d), lambda i: (i, 0)),
      out_shape=jax.ShapeDtypeStruct((m, d), jnp.float32),
  )(a0, a1, c, hr, b)


def kernel(x, edge_index, W1l, W1r, b1, W2l, W2r, b2):
  ei = edge_index.astype(jnp.int32)
  npad = E_PAD - N_EDGES
  pad_i = jnp.arange(npad, dtype=jnp.int32)
  pad_src = pad_i % N_NODES
  pad_dst = N_NODES + pad_i % (N_PAD - N_NODES)
  src = jnp.concatenate([ei[0], pad_src]).reshape(E_PAD // CHUNK, CHUNK)
  dst = jnp.concatenate([ei[1], pad_dst]).reshape(E_PAD // CHUNK, CHUNK)
  x_pad = jnp.pad(x, ((0, N_PAD - N_NODES), (0, 0)))

  wcat1 = jnp.concatenate([W1l.T, W1r.T], axis=1)
  pcat = _matmul(x_pad, wcat1)
  p1 = pcat[:, :D_HID]
  xr = pcat[:, D_HID:]
  p_split = jnp.concatenate([p1[:, :D_HALF], p1[:, D_HALF:]], axis=0)

  agg1, cnt = _agg_l1(p_split, src, dst)

  wcat2 = jnp.concatenate([W2l.T, W2r.T], axis=1)
  out2 = _layer_mid(agg1[0], agg1[1], cnt[0], xr, b1.reshape(1, -1), wcat2)
  p2 = out2[:, :D_OUT]
  hr = out2[:, D_OUT:]

  agg2 = _agg_l2(p2, src, dst)

  out = _layer_out(agg2[0], agg2[1], cnt[0], hr, b2.reshape(1, -1))
  return out[:N_NODES]


# speedup vs baseline: 14.1484x; 1.4884x over previous
import functools

import jax
import jax.numpy as jnp
from jax import lax
from jax.experimental import pallas as pl
from jax.experimental.pallas import tpu as pltpu
from jax.experimental.pallas import tpu_sc as plsc

N_NODES = 10000
N_EDGES = 160000
D_IN = 256
D_HID = 128
D_OUT = 64

NC = 2
NS = 16
NW = NC * NS

N_PAD = 10240
E_PAD = 163840
CHUNK = 128
ROWS_PER_TILE = N_PAD // NS
CNT_W = 8
D_HALF = 64


def _make_sc_agg(n_chunks, feature_split, with_cnt, NBUF):
  D = D_HALF
  mesh = plsc.VectorSubcoreMesh(core_axis_name="c", subcore_axis_name="s")
  out_type = [jax.ShapeDtypeStruct((N_PAD, 2 * D), jnp.float32)]
  if with_cnt:
    out_type.append(jax.ShapeDtypeStruct((N_PAD, CNT_W), jnp.float32))
  scratch = [
      pltpu.VMEM((n_chunks, CHUNK), jnp.int32),
      pltpu.VMEM((n_chunks, CHUNK), jnp.int32),
      [pltpu.VMEM((CHUNK, D), jnp.float32) for _ in range(NBUF)],
      pltpu.VMEM_SHARED((N_PAD, D), jnp.float32),
      [pltpu.SemaphoreType.DMA for _ in range(NBUF)],
      [pltpu.SemaphoreType.DMA for _ in range(NBUF)],
  ]
  if with_cnt:
    scratch += [
        pltpu.VMEM((CHUNK, CNT_W), jnp.float32),
        pltpu.VMEM_SHARED((N_PAD, CNT_W), jnp.float32),
    ]

  def body(p_hbm, src_hbm, dst_hbm, *rest):
    if with_cnt:
      (out_hbm, cnt_hbm, src_v, dst_v, rows, acc_sh, sem_g, sem_s, ones_v,
       cnt_sh) = rest
    else:
      out_hbm, src_v, dst_v, rows, acc_sh, sem_g, sem_s = rest
    cid = lax.axis_index("c")
    sid = lax.axis_index("s")
    row0 = sid * ROWS_PER_TILE
    if feature_split:
      chunk0 = sid * n_chunks
    else:
      chunk0 = (sid * NC + cid) * n_chunks

    pltpu.sync_copy(src_hbm.at[pl.ds(chunk0, n_chunks)], src_v)
    pltpu.sync_copy(dst_hbm.at[pl.ds(chunk0, n_chunks)], dst_v)
    half = cid if feature_split else 0

    def adjust(i, carry):
      for j in range(CHUNK // 16):
        sl = src_v[i, pl.ds(j * 16, 16)]
        src_v[i, pl.ds(j * 16, 16)] = sl + sl + half
      return carry
    lax.fori_loop(0, n_chunks, adjust, 0)

    def zero_row(i, carry):
      for j in range(D // 16):
        rows[0][i, pl.ds(j * 16, 16)] = jnp.zeros((16,), jnp.float32)
      return carry
    lax.fori_loop(0, CHUNK, zero_row, 0)
    for r in range(ROWS_PER_TILE // CHUNK):
      pltpu.sync_copy(rows[0], acc_sh.at[pl.ds(row0 + r * CHUNK, CHUNK)])
    if with_cnt:
      def zero_ones(i, carry):
        ones_v[i, :] = jnp.zeros((CNT_W,), jnp.float32)
        return carry
      lax.fori_loop(0, CHUNK, zero_ones, 0)
      for r in range(ROWS_PER_TILE // CHUNK):
        pltpu.sync_copy(ones_v, cnt_sh.at[pl.ds(row0 + r * CHUNK, CHUNK)])
      def fill_ones(i, carry):
        ones_v[i, :] = jnp.ones((CNT_W,), jnp.float32)
        return carry
      lax.fori_loop(0, CHUNK, fill_ones, 0)
    plsc.subcore_barrier()

    def gather(k, b):
      pltpu.async_copy(p_hbm.at[src_v.at[k]], rows[b], sem_g[b])

    def scatter(k, b):
      pltpu.async_copy(rows[b], acc_sh.at[dst_v.at[k]], sem_s[b], add=True)

    for b in range(NBUF - 1):
      gather(b, b)

    def step(j, carry):
      for b in range(NBUF):
        k = j * NBUF + b
        pltpu.make_async_copy(p_hbm.at[src_v.at[k]], rows[b],
                              sem_g[b]).wait()
        scatter(k, b)
        if with_cnt:
          pltpu.sync_copy(ones_v, cnt_sh.at[dst_v.at[k]], add=True)
        bn = (b + NBUF - 1) % NBUF
        kn = k + NBUF - 1

        @pl.when(k >= 1)
        def _wait_prev(bn=bn, kn=kn):
          pltpu.make_async_copy(rows[bn], acc_sh.at[dst_v.at[kn - NBUF]],
                                sem_s[bn]).wait()

        @pl.when(kn < n_chunks)
        def _prefetch(bn=bn, kn=kn):
          gather(kn, bn)
      return carry
    lax.fori_loop(0, n_chunks // NBUF, step, 0)
    bl = (n_chunks - 1) % NBUF
    pltpu.make_async_copy(rows[bl], acc_sh.at[dst_v.at[n_chunks - 1]],
                          sem_s[bl]).wait()

    plsc.subcore_barrier()
    col0 = cid * D
    pltpu.sync_copy(acc_sh.at[pl.ds(row0, ROWS_PER_TILE)],
                    out_hbm.at[pl.ds(row0, ROWS_PER_TILE), pl.ds(col0, D)])
    if with_cnt:
      @pl.when(cid == 0)
      def _cnt_out():
        pltpu.sync_copy(cnt_sh.at[pl.ds(row0, ROWS_PER_TILE)],
                        cnt_hbm.at[pl.ds(row0, ROWS_PER_TILE)])

  return pl.kernel(body, out_type=out_type, mesh=mesh, scratch_types=scratch,
                   compiler_params=pltpu.CompilerParams(
                       use_tc_tiling_on_sc=False))


_make_sc_agg = functools.lru_cache(maxsize=None)(_make_sc_agg)


def _agg_l1(p_tab, src, dst):
  return _make_sc_agg(E_PAD // CHUNK // NS, True, True, 5)(p_tab, src, dst)


def _agg_l2(p_tab, src, dst):
  out = _make_sc_agg(E_PAD // CHUNK // NW, False, False, 8)(p_tab, src, dst)
  if isinstance(out, (list, tuple)):
    out = out[0]
  return out


_BR = 1024


def _proj1_body(x_ref, w_ref, p_ref, xr_ref):
  w = w_ref[...]
  x = x_ref[...]
  p_ref[...] = jnp.dot(x, w[:, :D_HID], preferred_element_type=jnp.float32)
  xr_ref[...] = jnp.dot(x, w[:, D_HID:], preferred_element_type=jnp.float32)


def _proj1(x, w):
  k = x.shape[1]
  n = w.shape[1]
  return pl.pallas_call(
      _proj1_body,
      grid=(N_PAD // _BR,),
      in_specs=[
          pl.BlockSpec((_BR, k), lambda i: (i, 0)),
          pl.BlockSpec((k, n), lambda i: (0, 0)),
      ],
      out_specs=[
          pl.BlockSpec((_BR, D_HID), lambda i: (i, 0)),
          pl.BlockSpec((_BR, D_HID), lambda i: (i, 0)),
      ],
      out_shape=[
          jax.ShapeDtypeStruct((N_PAD, D_HID), jnp.float32),
          jax.ShapeDtypeStruct((N_PAD, D_HID), jnp.float32),
      ],
  )(x, w)


def _mid_body(s_ref, c_ref, xr_ref, b_ref, w_ref, o_ref):
  rcnt = 1.0 / jnp.maximum(c_ref[:, 0:1], 1.0)
  h = s_ref[...] * rcnt + xr_ref[...] + b_ref[...]
  h = jnp.maximum(h, 0.0)
  o_ref[...] = jnp.dot(h, w_ref[...], preferred_element_type=jnp.float32)


def _layer_mid(s, c, xr, b, w):
  m, d = s.shape
  n = w.shape[1]
  return pl.pallas_call(
      _mid_body,
      grid=(m // _BR,),
      in_specs=[
          pl.BlockSpec((_BR, d), lambda i: (i, 0)),
          pl.BlockSpec((_BR, CNT_W), lambda i: (i, 0)),
          pl.BlockSpec((_BR, d), lambda i: (i, 0)),
          pl.BlockSpec((1, d), lambda i: (0, 0)),
          pl.BlockSpec((d, n), lambda i: (0, 0)),
      ],
      out_specs=pl.BlockSpec((_BR, n), lambda i: (i, 0)),
      out_shape=jax.ShapeDtypeStruct((m, n), jnp.float32),
  )(s, c, xr, b, w)


def _out_body(a_ref, c_ref, h2_ref, b_ref, o_ref):
  rcnt = 1.0 / jnp.maximum(c_ref[:, 0:1], 1.0)
  s2 = a_ref[:, :D_OUT] + a_ref[:, D_OUT:]
  o = s2 * rcnt + h2_ref[:, D_OUT:] + b_ref[...]
  m = jnp.max(o, axis=-1, keepdims=True)
  e = jnp.exp(o - m)
  lse = m + jnp.log(jnp.sum(e, axis=-1, keepdims=True))
  o_ref[...] = o - lse


def _layer_out(a, c, h2, b):
  m, d = a.shape
  return pl.pallas_call(
      _out_body,
      grid=(m // _BR,),
      in_specs=[
          pl.BlockSpec((_BR, d), lambda i: (i, 0)),
          pl.BlockSpec((_BR, CNT_W), lambda i: (i, 0)),
          pl.BlockSpec((_BR, d), lambda i: (i, 0)),
          pl.BlockSpec((1, D_OUT), lambda i: (0, 0)),
      ],
      out_specs=pl.BlockSpec((_BR, D_OUT), lambda i: (i, 0)),
      out_shape=jax.ShapeDtypeStruct((N_NODES, D_OUT), jnp.float32),
  )(a, c, h2, b)


def kernel(x, edge_index, W1l, W1r, b1, W2l, W2r, b2):
  ei = edge_index.astype(jnp.int32)
  npad = E_PAD - N_EDGES
  pad_i = jnp.arange(npad, dtype=jnp.int32)
  pad_src = pad_i % N_NODES
  pad_dst = N_NODES + pad_i % (N_PAD - N_NODES)
  src = jnp.concatenate([ei[0], pad_src]).reshape(E_PAD // CHUNK, CHUNK)
  dst = jnp.concatenate([ei[1], pad_dst]).reshape(E_PAD // CHUNK, CHUNK)

  wcat1 = jnp.concatenate([W1l.T, W1r.T], axis=1)
  p1, xr = _proj1(x, wcat1)

  agg1, cnt = _agg_l1(p1.reshape(2 * N_PAD, D_HALF), src, dst)

  wcat2 = jnp.concatenate([W2l.T, W2r.T], axis=1)
  out2 = _layer_mid(agg1, cnt, xr, b1.reshape(1, -1), wcat2)

  agg2 = _agg_l2(out2.reshape(2 * N_PAD, D_HALF), src, dst)

  return _layer_out(agg2, cnt, out2, b2.reshape(1, -1))


# speedup vs baseline: 14.8985x; 1.0530x over previous
import functools

import jax
import jax.numpy as jnp
from jax import lax
from jax.experimental import pallas as pl
from jax.experimental.pallas import tpu as pltpu
from jax.experimental.pallas import tpu_sc as plsc

N_NODES = 10000
N_EDGES = 160000
D_IN = 256
D_HID = 128
D_OUT = 64

NC = 2
NS = 16
NW = NC * NS

N_PAD = 10240
E_PAD = 163840
CHUNK = 128
ROWS_PER_TILE = N_PAD // NS
CNT_W = 8
D_HALF = 64


def _make_sc_agg(n_chunks, feature_split, with_cnt, NBUF):
  D = D_HALF
  mesh = plsc.VectorSubcoreMesh(core_axis_name="c", subcore_axis_name="s")
  out_type = [jax.ShapeDtypeStruct((N_PAD, 2 * D), jnp.float32)]
  if with_cnt:
    out_type.append(jax.ShapeDtypeStruct((N_PAD, 2 * D), jnp.float32))
  scratch = [
      pltpu.VMEM((n_chunks, CHUNK), jnp.int32),
      pltpu.VMEM((n_chunks, CHUNK), jnp.int32),
      [pltpu.VMEM((CHUNK, D), jnp.float32) for _ in range(NBUF)],
      pltpu.VMEM_SHARED((N_PAD, D), jnp.float32),
      [pltpu.SemaphoreType.DMA for _ in range(NBUF)],
      [pltpu.SemaphoreType.DMA for _ in range(NBUF)],
  ]
  if with_cnt:
    scratch += [
        pltpu.VMEM((CHUNK, CNT_W), jnp.float32),
        pltpu.VMEM_SHARED((N_PAD, CNT_W), jnp.float32),
    ]

  def body(p_hbm, edges_hbm, *rest):
    if with_cnt:
      (out_hbm, cnt_hbm, src_v, dst_v, rows, acc_sh, sem_g, sem_s, ones_v,
       cnt_sh) = rest
    else:
      out_hbm, src_v, dst_v, rows, acc_sh, sem_g, sem_s = rest
    cid = lax.axis_index("c")
    sid = lax.axis_index("s")
    row0 = sid * ROWS_PER_TILE
    if feature_split:
      chunk0 = sid * n_chunks
    else:
      chunk0 = (sid * NC + cid) * n_chunks

    pltpu.sync_copy(edges_hbm.at[0, pl.ds(chunk0, n_chunks)], src_v)
    pltpu.sync_copy(edges_hbm.at[1, pl.ds(chunk0, n_chunks)], dst_v)
    half = cid if feature_split else 0

    def adjust(i, carry):
      for j in range(CHUNK // 16):
        sl = src_v[i, pl.ds(j * 16, 16)]
        src_v[i, pl.ds(j * 16, 16)] = sl + sl + half
      return carry
    lax.fori_loop(0, n_chunks, adjust, 0)

    def zero_row(i, carry):
      for j in range(D // 16):
        rows[0][i, pl.ds(j * 16, 16)] = jnp.zeros((16,), jnp.float32)
      return carry
    lax.fori_loop(0, CHUNK, zero_row, 0)
    for r in range(ROWS_PER_TILE // CHUNK):
      pltpu.sync_copy(rows[0], acc_sh.at[pl.ds(row0 + r * CHUNK, CHUNK)])
    if with_cnt:
      def zero_ones(i, carry):
        ones_v[i, :] = jnp.zeros((CNT_W,), jnp.float32)
        return carry
      lax.fori_loop(0, CHUNK, zero_ones, 0)
      for r in range(ROWS_PER_TILE // CHUNK):
        pltpu.sync_copy(ones_v, cnt_sh.at[pl.ds(row0 + r * CHUNK, CHUNK)])
      def fill_ones(i, carry):
        ones_v[i, :] = jnp.ones((CNT_W,), jnp.float32)
        return carry
      lax.fori_loop(0, CHUNK, fill_ones, 0)
    plsc.subcore_barrier()

    def gather(k, b):
      pltpu.async_copy(p_hbm.at[src_v.at[k]], rows[b], sem_g[b])

    def scatter(k, b):
      pltpu.async_copy(rows[b], acc_sh.at[dst_v.at[k]], sem_s[b], add=True)

    for b in range(NBUF - 1):
      gather(b, b)

    def step(j, carry):
      for b in range(NBUF):
        k = j * NBUF + b
        pltpu.make_async_copy(p_hbm.at[src_v.at[k]], rows[b],
                              sem_g[b]).wait()
        scatter(k, b)
        if with_cnt:
          pltpu.sync_copy(ones_v, cnt_sh.at[dst_v.at[k]], add=True)
        bn = (b + NBUF - 1) % NBUF
        kn = k + NBUF - 1

        @pl.when(k >= 1)
        def _wait_prev(bn=bn, kn=kn):
          pltpu.make_async_copy(rows[bn], acc_sh.at[dst_v.at[kn - NBUF]],
                                sem_s[bn]).wait()

        @pl.when(kn < n_chunks)
        def _prefetch(bn=bn, kn=kn):
          gather(kn, bn)
      return carry
    lax.fori_loop(0, n_chunks // NBUF, step, 0)
    bl = (n_chunks - 1) % NBUF
    pltpu.make_async_copy(rows[bl], acc_sh.at[dst_v.at[n_chunks - 1]],
                          sem_s[bl]).wait()

    plsc.subcore_barrier()
    col0 = cid * D
    pltpu.sync_copy(acc_sh.at[pl.ds(row0, ROWS_PER_TILE)],
                    out_hbm.at[pl.ds(row0, ROWS_PER_TILE), pl.ds(col0, D)])
    if with_cnt:
      @pl.when(cid == 0)
      def _cnt_out():
        pltpu.sync_copy(cnt_sh.at[pl.ds(row0, ROWS_PER_TILE)],
                        cnt_hbm.at[pl.ds(row0, ROWS_PER_TILE),
                                   pl.ds(0, CNT_W)])

  return pl.kernel(body, out_type=out_type, mesh=mesh, scratch_types=scratch,
                   compiler_params=pltpu.CompilerParams(
                       use_tc_tiling_on_sc=False))


_make_sc_agg = functools.lru_cache(maxsize=None)(_make_sc_agg)


def _agg_l1(p_tab, edges):
  return _make_sc_agg(E_PAD // CHUNK // NS, True, True, 5)(p_tab, edges)


def _agg_l2(p_tab, edges):
  out = _make_sc_agg(E_PAD // CHUNK // NW, False, False, 8)(p_tab, edges)
  if isinstance(out, (list, tuple)):
    out = out[0]
  return out


_BR = 1024


def _proj1_body(x_ref, w_ref, p_ref, xr_ref):
  w = w_ref[...].astype(jnp.bfloat16)
  x = x_ref[...].astype(jnp.bfloat16)
  p_ref[...] = jnp.dot(x, w[:, :D_HID], preferred_element_type=jnp.float32)
  xr_ref[...] = jnp.dot(x, w[:, D_HID:], preferred_element_type=jnp.float32)


def _proj1(x, w):
  k = x.shape[1]
  n = w.shape[1]
  return pl.pallas_call(
      _proj1_body,
      grid=(N_PAD // _BR,),
      in_specs=[
          pl.BlockSpec((_BR, k), lambda i: (i, 0)),
          pl.BlockSpec((k, n), lambda i: (0, 0)),
      ],
      out_specs=[
          pl.BlockSpec((_BR, D_HID), lambda i: (i, 0)),
          pl.BlockSpec((_BR, D_HID), lambda i: (i, 0)),
      ],
      out_shape=[
          jax.ShapeDtypeStruct((N_PAD, D_HID), jnp.float32),
          jax.ShapeDtypeStruct((N_PAD, D_HID), jnp.float32),
      ],
  )(x, w)


def _mid_body(s_ref, c_ref, xr_ref, b_ref, w_ref, o_ref):
  rcnt = 1.0 / jnp.maximum(c_ref[:, 0:1], 1.0)
  h = s_ref[...] * rcnt + xr_ref[...] + b_ref[...]
  h = jnp.maximum(h, 0.0).astype(jnp.bfloat16)
  o_ref[...] = jnp.dot(h, w_ref[...].astype(jnp.bfloat16),
                       preferred_element_type=jnp.float32)


def _layer_mid(s, c, xr, b, w):
  m, d = s.shape
  n = w.shape[1]
  return pl.pallas_call(
      _mid_body,
      grid=(m // _BR,),
      in_specs=[
          pl.BlockSpec((_BR, d), lambda i: (i, 0)),
          pl.BlockSpec((_BR, d), lambda i: (i, 0)),
          pl.BlockSpec((_BR, d), lambda i: (i, 0)),
          pl.BlockSpec((1, d), lambda i: (0, 0)),
          pl.BlockSpec((d, n), lambda i: (0, 0)),
      ],
      out_specs=pl.BlockSpec((_BR, n), lambda i: (i, 0)),
      out_shape=jax.ShapeDtypeStruct((m, n), jnp.float32),
  )(s, c, xr, b, w)


def _out_body(a_ref, c_ref, h2_ref, b_ref, o_ref):
  rcnt = 1.0 / jnp.maximum(c_ref[:, 0:1], 1.0)
  s2 = a_ref[:, :D_OUT] + a_ref[:, D_OUT:]
  o = s2 * rcnt + h2_ref[:, D_OUT:] + b_ref[...]
  m = jnp.max(o, axis=-1, keepdims=True)
  e = jnp.exp(o - m)
  lse = m + jnp.log(jnp.sum(e, axis=-1, keepdims=True))
  o_ref[...] = o - lse


def _layer_out(a, c, h2, b):
  m, d = a.shape
  return pl.pallas_call(
      _out_body,
      grid=(m // _BR,),
      in_specs=[
          pl.BlockSpec((_BR, d), lambda i: (i, 0)),
          pl.BlockSpec((_BR, d), lambda i: (i, 0)),
          pl.BlockSpec((_BR, d), lambda i: (i, 0)),
          pl.BlockSpec((1, D_OUT), lambda i: (0, 0)),
      ],
      out_specs=pl.BlockSpec((_BR, D_OUT), lambda i: (i, 0)),
      out_shape=jax.ShapeDtypeStruct((N_NODES, D_OUT), jnp.float32),
  )(a, c, h2, b)


def kernel(x, edge_index, W1l, W1r, b1, W2l, W2r, b2):
  ei = edge_index.astype(jnp.int32)
  npad = E_PAD - N_EDGES
  pad_i = jnp.arange(npad, dtype=jnp.int32)
  pad_src = pad_i % N_NODES
  pad_dst = N_NODES + pad_i % (N_PAD - N_NODES)
  edges = jnp.concatenate(
      [ei, jnp.stack([pad_src, pad_dst])], axis=1
  ).reshape(2, E_PAD // CHUNK, CHUNK)

  wcat1 = jnp.concatenate([W1l.T, W1r.T], axis=1)
  p1, xr = _proj1(x, wcat1)

  agg1, cnt = _agg_l1(p1.reshape(2 * N_PAD, D_HALF), edges)

  wcat2 = jnp.concatenate([W2l.T, W2r.T], axis=1)
  out2 = _layer_mid(agg1, cnt, xr, b1.reshape(1, -1), wcat2)

  agg2 = _agg_l2(out2.reshape(2 * N_PAD, D_HALF), edges)

  return _layer_out(agg2, cnt, out2, b2.reshape(1, -1))


# speedup vs baseline: 15.0775x; 1.0120x over previous
import functools

import jax
import jax.numpy as jnp
from jax import lax
from jax.experimental import pallas as pl
from jax.experimental.pallas import tpu as pltpu
from jax.experimental.pallas import tpu_sc as plsc

N_NODES = 10000
N_EDGES = 160000
D_IN = 256
D_HID = 128
D_OUT = 64

NC = 2
NS = 16
NW = NC * NS

N_PAD = 10240
E_PAD = 163840
CHUNK = 128
ROWS_PER_TILE = N_PAD // NS
CNT_W = 8
D_HALF = 64


def _make_sc_agg(n_chunks, feature_split, with_cnt, NBUF):
  D = D_HALF
  mesh = plsc.VectorSubcoreMesh(core_axis_name="c", subcore_axis_name="s")
  out_type = [jax.ShapeDtypeStruct((N_PAD, 2 * D), jnp.float32)]
  if with_cnt:
    out_type.append(jax.ShapeDtypeStruct((N_PAD, 2 * D), jnp.float32))
  scratch = [
      pltpu.VMEM((n_chunks, CHUNK), jnp.int32),
      pltpu.VMEM((n_chunks, CHUNK), jnp.int32),
      [pltpu.VMEM((CHUNK, D), jnp.float32) for _ in range(NBUF)],
      pltpu.VMEM_SHARED((N_PAD, D), jnp.float32),
      [pltpu.SemaphoreType.DMA for _ in range(NBUF)],
      [pltpu.SemaphoreType.DMA for _ in range(NBUF)],
  ]
  if with_cnt:
    scratch += [
        pltpu.VMEM((CHUNK, CNT_W), jnp.float32),
        pltpu.VMEM_SHARED((N_PAD, CNT_W), jnp.float32),
    ]

  def body(p_hbm, edges_hbm, *rest):
    if with_cnt:
      (out_hbm, cnt_hbm, src_v, dst_v, rows, acc_sh, sem_g, sem_s, ones_v,
       cnt_sh) = rest
    else:
      out_hbm, src_v, dst_v, rows, acc_sh, sem_g, sem_s = rest
    cid = lax.axis_index("c")
    sid = lax.axis_index("s")
    row0 = sid * ROWS_PER_TILE
    if feature_split:
      chunk0 = sid * n_chunks
    else:
      chunk0 = (sid * NC + cid) * n_chunks

    pltpu.sync_copy(edges_hbm.at[0, pl.ds(chunk0, n_chunks)], src_v)
    pltpu.sync_copy(edges_hbm.at[1, pl.ds(chunk0, n_chunks)], dst_v)
    if feature_split:
      half = cid

      def adjust(i, carry):
        for j in range(CHUNK // 16):
          sl = src_v[i, pl.ds(j * 16, 16)]
          src_v[i, pl.ds(j * 16, 16)] = sl + sl + half
        return carry
      lax.fori_loop(0, n_chunks, adjust, 0)

    def zero_row(i, carry):
      for j in range(D // 16):
        rows[0][i, pl.ds(j * 16, 16)] = jnp.zeros((16,), jnp.float32)
      return carry
    lax.fori_loop(0, CHUNK, zero_row, 0)
    for r in range(ROWS_PER_TILE // CHUNK):
      pltpu.sync_copy(rows[0], acc_sh.at[pl.ds(row0 + r * CHUNK, CHUNK)])
    if with_cnt:
      def zero_ones(i, carry):
        ones_v[i, :] = jnp.zeros((CNT_W,), jnp.float32)
        return carry
      lax.fori_loop(0, CHUNK, zero_ones, 0)
      for r in range(ROWS_PER_TILE // CHUNK):
        pltpu.sync_copy(ones_v, cnt_sh.at[pl.ds(row0 + r * CHUNK, CHUNK)])
      def fill_ones(i, carry):
        ones_v[i, :] = jnp.ones((CNT_W,), jnp.float32)
        return carry
      lax.fori_loop(0, CHUNK, fill_ones, 0)
    plsc.subcore_barrier()

    def gather(k, b):
      pltpu.async_copy(p_hbm.at[src_v.at[k]], rows[b], sem_g[b])

    def scatter(k, b):
      pltpu.async_copy(rows[b], acc_sh.at[dst_v.at[k]], sem_s[b], add=True)

    for b in range(NBUF - 1):
      gather(b, b)

    def step(j, carry):
      for b in range(NBUF):
        k = j * NBUF + b
        pltpu.make_async_copy(p_hbm.at[src_v.at[k]], rows[b],
                              sem_g[b]).wait()
        scatter(k, b)
        if with_cnt:
          pltpu.sync_copy(ones_v, cnt_sh.at[dst_v.at[k]], add=True)
        bn = (b + NBUF - 1) % NBUF
        kn = k + NBUF - 1

        @pl.when(k >= 1)
        def _wait_prev(bn=bn, kn=kn):
          pltpu.make_async_copy(rows[bn], acc_sh.at[dst_v.at[kn - NBUF]],
                                sem_s[bn]).wait()

        @pl.when(kn < n_chunks)
        def _prefetch(bn=bn, kn=kn):
          gather(kn, bn)
      return carry
    lax.fori_loop(0, n_chunks // NBUF, step, 0)
    bl = (n_chunks - 1) % NBUF
    pltpu.make_async_copy(rows[bl], acc_sh.at[dst_v.at[n_chunks - 1]],
                          sem_s[bl]).wait()

    plsc.subcore_barrier()
    col0 = cid * D
    pltpu.sync_copy(acc_sh.at[pl.ds(row0, ROWS_PER_TILE)],
                    out_hbm.at[pl.ds(row0, ROWS_PER_TILE), pl.ds(col0, D)])
    if with_cnt:
      @pl.when(cid == 0)
      def _cnt_out():
        pltpu.sync_copy(cnt_sh.at[pl.ds(row0, ROWS_PER_TILE)],
                        cnt_hbm.at[pl.ds(row0, ROWS_PER_TILE),
                                   pl.ds(0, CNT_W)])

  return pl.kernel(body, out_type=out_type, mesh=mesh, scratch_types=scratch,
                   compiler_params=pltpu.CompilerParams(
                       use_tc_tiling_on_sc=False))


_make_sc_agg = functools.lru_cache(maxsize=None)(_make_sc_agg)


def _agg_l1(p_tab, edges):
  return _make_sc_agg(E_PAD // CHUNK // NS, True, True, 5)(p_tab, edges)


def _agg_l2(p_tab, edges):
  out = _make_sc_agg(E_PAD // CHUNK // NW, False, False, 8)(p_tab, edges)
  if isinstance(out, (list, tuple)):
    out = out[0]
  return out


_BR = 2048


def _proj1_body(x_ref, w_ref, p_ref, xr_ref):
  w = w_ref[...].astype(jnp.bfloat16)
  x = x_ref[...].astype(jnp.bfloat16)
  p_ref[...] = jnp.dot(x, w[:, :D_HID], preferred_element_type=jnp.float32)
  xr_ref[...] = jnp.dot(x, w[:, D_HID:], preferred_element_type=jnp.float32)


def _proj1(x, w):
  k = x.shape[1]
  n = w.shape[1]
  return pl.pallas_call(
      _proj1_body,
      grid=(N_PAD // _BR,),
      in_specs=[
          pl.BlockSpec((_BR, k), lambda i: (i, 0)),
          pl.BlockSpec((k, n), lambda i: (0, 0)),
      ],
      out_specs=[
          pl.BlockSpec((_BR, D_HID), lambda i: (i, 0)),
          pl.BlockSpec((_BR, D_HID), lambda i: (i, 0)),
      ],
      out_shape=[
          jax.ShapeDtypeStruct((N_PAD, D_HID), jnp.float32),
          jax.ShapeDtypeStruct((N_PAD, D_HID), jnp.float32),
      ],
  )(x, w)


def _mid_body(s_ref, c_ref, xr_ref, b_ref, w_ref, o_ref):
  rcnt = 1.0 / jnp.maximum(c_ref[:, 0:1], 1.0)
  h = s_ref[...] * rcnt + xr_ref[...] + b_ref[...]
  h = jnp.maximum(h, 0.0).astype(jnp.bfloat16)
  o_ref[...] = jnp.dot(h, w_ref[...].astype(jnp.bfloat16),
                       preferred_element_type=jnp.float32)


def _layer_mid(s, c, xr, b, w):
  m, d = s.shape
  n = w.shape[1]
  return pl.pallas_call(
      _mid_body,
      grid=(m // _BR,),
      in_specs=[
          pl.BlockSpec((_BR, d), lambda i: (i, 0)),
          pl.BlockSpec((_BR, d), lambda i: (i, 0)),
          pl.BlockSpec((_BR, d), lambda i: (i, 0)),
          pl.BlockSpec((1, d), lambda i: (0, 0)),
          pl.BlockSpec((d, n), lambda i: (0, 0)),
      ],
      out_specs=pl.BlockSpec((_BR, n), lambda i: (i, 0)),
      out_shape=jax.ShapeDtypeStruct((m, n), jnp.float32),
  )(s, c, xr, b, w)


def _out_body(a_ref, c_ref, h2_ref, b_ref, o_ref):
  rcnt = 1.0 / jnp.maximum(c_ref[:, 0:1], 1.0)
  s2 = a_ref[:, :D_OUT] + a_ref[:, D_OUT:]
  o = s2 * rcnt + h2_ref[:, D_OUT:] + b_ref[...]
  m = jnp.max(o, axis=-1, keepdims=True)
  e = jnp.exp(o - m)
  lse = m + jnp.log(jnp.sum(e, axis=-1, keepdims=True))
  o_ref[...] = o - lse


def _layer_out(a, c, h2, b):
  m, d = a.shape
  return pl.pallas_call(
      _out_body,
      grid=(m // _BR,),
      in_specs=[
          pl.BlockSpec((_BR, d), lambda i: (i, 0)),
          pl.BlockSpec((_BR, d), lambda i: (i, 0)),
          pl.BlockSpec((_BR, d), lambda i: (i, 0)),
          pl.BlockSpec((1, D_OUT), lambda i: (0, 0)),
      ],
      out_specs=pl.BlockSpec((_BR, D_OUT), lambda i: (i, 0)),
      out_shape=jax.ShapeDtypeStruct((N_NODES, D_OUT), jnp.float32),
  )(a, c, h2, b)


def kernel(x, edge_index, W1l, W1r, b1, W2l, W2r, b2):
  ei = edge_index.astype(jnp.int32)
  npad = E_PAD - N_EDGES
  pad_i = jnp.arange(npad, dtype=jnp.int32)
  pad_src = pad_i % N_NODES
  pad_dst = N_NODES + pad_i % (N_PAD - N_NODES)
  edges = jnp.concatenate(
      [ei, jnp.stack([pad_src, pad_dst])], axis=1
  ).reshape(2, E_PAD // CHUNK, CHUNK)

  wcat1 = jnp.concatenate([W1l.T, W1r.T], axis=1)
  p1, xr = _proj1(x, wcat1)

  agg1, cnt = _agg_l1(p1.reshape(2 * N_PAD, D_HALF), edges)

  wcat2 = jnp.concatenate([W2l.T, W2r.T], axis=1)
  out2 = _layer_mid(agg1, cnt, xr, b1.reshape(1, -1), wcat2)

  agg2 = _agg_l2(out2[:, :D_OUT], edges)

  return _layer_out(agg2, cnt, out2, b2.reshape(1, -1))


# speedup vs baseline: 15.5115x; 1.0288x over previous
import functools

import jax
import jax.numpy as jnp
from jax import lax
from jax.experimental import pallas as pl
from jax.experimental.pallas import tpu as pltpu
from jax.experimental.pallas import tpu_sc as plsc

N_NODES = 10000
N_EDGES = 160000
D_IN = 256
D_HID = 128
D_OUT = 64

NC = 2
NS = 16
NW = NC * NS

N_PAD = 10240
E_PAD = 163840
CHUNK = 128
ROWS_PER_TILE = N_PAD // NS
CNT_W = 8
D_HALF = 64


def _make_sc_agg(n_chunks, feature_split, with_cnt, NBUF):
  D = D_HALF
  mesh = plsc.VectorSubcoreMesh(core_axis_name="c", subcore_axis_name="s")
  out_type = [jax.ShapeDtypeStruct((N_PAD, 2 * D), jnp.float32)]
  if with_cnt:
    out_type.append(jax.ShapeDtypeStruct((N_PAD, 2 * D), jnp.float32))
  scratch = [
      pltpu.VMEM((n_chunks, CHUNK), jnp.int32),
      pltpu.VMEM((n_chunks, CHUNK), jnp.int32),
      [pltpu.VMEM((CHUNK, D), jnp.float32) for _ in range(NBUF)],
      pltpu.VMEM_SHARED((N_PAD, D), jnp.float32),
      [pltpu.SemaphoreType.DMA for _ in range(NBUF)],
      [pltpu.SemaphoreType.DMA for _ in range(NBUF)],
  ]
  if with_cnt:
    scratch += [
        pltpu.VMEM((CHUNK, CNT_W), jnp.float32),
        pltpu.VMEM_SHARED((N_PAD, CNT_W), jnp.float32),
    ]

  def body(p_hbm, edges_hbm, *rest):
    if with_cnt:
      (out_hbm, cnt_hbm, src_v, dst_v, rows, acc_sh, sem_g, sem_s, ones_v,
       cnt_sh) = rest
    else:
      out_hbm, src_v, dst_v, rows, acc_sh, sem_g, sem_s = rest
    cid = lax.axis_index("c")
    sid = lax.axis_index("s")
    row0 = sid * ROWS_PER_TILE
    if feature_split:
      chunk0 = sid * n_chunks
    else:
      chunk0 = (sid * NC + cid) * n_chunks

    pltpu.sync_copy(edges_hbm.at[0, pl.ds(chunk0, n_chunks)], src_v)
    pltpu.sync_copy(edges_hbm.at[1, pl.ds(chunk0, n_chunks)], dst_v)
    half = cid if feature_split else 0

    def adjust(i, carry):
      for j in range(CHUNK // 16):
        sl = src_v[i, pl.ds(j * 16, 16)]
        src_v[i, pl.ds(j * 16, 16)] = sl + sl + half
      return carry
    lax.fori_loop(0, n_chunks, adjust, 0)

    def zero_row(i, carry):
      for j in range(D // 16):
        rows[0][i, pl.ds(j * 16, 16)] = jnp.zeros((16,), jnp.float32)
      return carry
    lax.fori_loop(0, CHUNK, zero_row, 0)
    for r in range(ROWS_PER_TILE // CHUNK):
      pltpu.sync_copy(rows[0], acc_sh.at[pl.ds(row0 + r * CHUNK, CHUNK)])
    if with_cnt:
      def zero_ones(i, carry):
        ones_v[i, :] = jnp.zeros((CNT_W,), jnp.float32)
        return carry
      lax.fori_loop(0, CHUNK, zero_ones, 0)
      for r in range(ROWS_PER_TILE // CHUNK):
        pltpu.sync_copy(ones_v, cnt_sh.at[pl.ds(row0 + r * CHUNK, CHUNK)])
      def fill_ones(i, carry):
        ones_v[i, :] = jnp.ones((CNT_W,), jnp.float32)
        return carry
      lax.fori_loop(0, CHUNK, fill_ones, 0)
    plsc.subcore_barrier()

    def gather(k, b):
      pltpu.async_copy(p_hbm.at[src_v.at[k]], rows[b], sem_g[b])

    def scatter(k, b):
      pltpu.async_copy(rows[b], acc_sh.at[dst_v.at[k]], sem_s[b], add=True)

    for b in range(NBUF - 1):
      gather(b, b)

    def step(j, carry):
      for b in range(NBUF):
        k = j * NBUF + b
        pltpu.make_async_copy(p_hbm.at[src_v.at[k]], rows[b],
                              sem_g[b]).wait()
        scatter(k, b)
        if with_cnt:
          pltpu.sync_copy(ones_v, cnt_sh.at[dst_v.at[k]], add=True)
        bn = (b + NBUF - 1) % NBUF
        kn = k + NBUF - 1

        @pl.when(k >= 1)
        def _wait_prev(bn=bn, kn=kn):
          pltpu.make_async_copy(rows[bn], acc_sh.at[dst_v.at[kn - NBUF]],
                                sem_s[bn]).wait()

        @pl.when(kn < n_chunks)
        def _prefetch(bn=bn, kn=kn):
          gather(kn, bn)
      return carry
    lax.fori_loop(0, n_chunks // NBUF, step, 0)
    bl = (n_chunks - 1) % NBUF
    pltpu.make_async_copy(rows[bl], acc_sh.at[dst_v.at[n_chunks - 1]],
                          sem_s[bl]).wait()

    plsc.subcore_barrier()
    col0 = cid * D
    pltpu.sync_copy(acc_sh.at[pl.ds(row0, ROWS_PER_TILE)],
                    out_hbm.at[pl.ds(row0, ROWS_PER_TILE), pl.ds(col0, D)])
    if with_cnt:
      @pl.when(cid == 0)
      def _cnt_out():
        pltpu.sync_copy(cnt_sh.at[pl.ds(row0, ROWS_PER_TILE)],
                        cnt_hbm.at[pl.ds(row0, ROWS_PER_TILE),
                                   pl.ds(0, CNT_W)])

  return pl.kernel(body, out_type=out_type, mesh=mesh, scratch_types=scratch,
                   compiler_params=pltpu.CompilerParams(
                       use_tc_tiling_on_sc=False))


_make_sc_agg = functools.lru_cache(maxsize=None)(_make_sc_agg)


def _agg_l1(p_tab, edges):
  return _make_sc_agg(E_PAD // CHUNK // NS, True, True, 5)(p_tab, edges)


def _agg_l2(p_tab, edges):
  out = _make_sc_agg(E_PAD // CHUNK // NW, False, False, 8)(p_tab, edges)
  if isinstance(out, (list, tuple)):
    out = out[0]
  return out


_BR = 2048


def _proj1_body(x_ref, w_ref, p_ref, xr_ref):
  w = w_ref[...].astype(jnp.bfloat16)
  x = x_ref[...].astype(jnp.bfloat16)
  p_ref[...] = jnp.dot(x, w[:, :D_HID], preferred_element_type=jnp.float32)
  xr_ref[...] = jnp.dot(x, w[:, D_HID:], preferred_element_type=jnp.float32)


def _proj1(x, w):
  k = x.shape[1]
  n = w.shape[1]
  return pl.pallas_call(
      _proj1_body,
      grid=(N_PAD // _BR,),
      in_specs=[
          pl.BlockSpec((_BR, k), lambda i: (i, 0)),
          pl.BlockSpec((k, n), lambda i: (0, 0)),
      ],
      out_specs=[
          pl.BlockSpec((_BR, D_HID), lambda i: (i, 0)),
          pl.BlockSpec((_BR, D_HID), lambda i: (i, 0)),
      ],
      out_shape=[
          jax.ShapeDtypeStruct((N_PAD, D_HID), jnp.float32),
          jax.ShapeDtypeStruct((N_PAD, D_HID), jnp.float32),
      ],
  )(x, w)


def _mid_body(s_ref, c_ref, xr_ref, b_ref, w_ref, o_ref):
  rcnt = 1.0 / jnp.maximum(c_ref[:, 0:1], 1.0)
  h = s_ref[...] * rcnt + xr_ref[...] + b_ref[...]
  h = jnp.maximum(h, 0.0).astype(jnp.bfloat16)
  o_ref[...] = jnp.dot(h, w_ref[...].astype(jnp.bfloat16),
                       preferred_element_type=jnp.float32)


def _layer_mid(s, c, xr, b, w):
  m, d = s.shape
  n = w.shape[1]
  return pl.pallas_call(
      _mid_body,
      grid=(m // _BR,),
      in_specs=[
          pl.BlockSpec((_BR, d), lambda i: (i, 0)),
          pl.BlockSpec((_BR, d), lambda i: (i, 0)),
          pl.BlockSpec((_BR, d), lambda i: (i, 0)),
          pl.BlockSpec((1, d), lambda i: (0, 0)),
          pl.BlockSpec((d, n), lambda i: (0, 0)),
      ],
      out_specs=pl.BlockSpec((_BR, n), lambda i: (i, 0)),
      out_shape=jax.ShapeDtypeStruct((m, n), jnp.float32),
  )(s, c, xr, b, w)


def _out_body(a_ref, c_ref, h2_ref, b_ref, o_ref):
  rcnt = 1.0 / jnp.maximum(c_ref[:, 0:1], 1.0)
  s2 = a_ref[:, :D_OUT] + a_ref[:, D_OUT:]
  o = s2 * rcnt + h2_ref[:, D_OUT:] + b_ref[...]
  m = jnp.max(o, axis=-1, keepdims=True)
  e = jnp.exp(o - m)
  lse = m + jnp.log(jnp.sum(e, axis=-1, keepdims=True))
  o_ref[...] = o - lse


def _layer_out(a, c, h2, b):
  m, d = a.shape
  return pl.pallas_call(
      _out_body,
      grid=(m // _BR,),
      in_specs=[
          pl.BlockSpec((_BR, d), lambda i: (i, 0)),
          pl.BlockSpec((_BR, d), lambda i: (i, 0)),
          pl.BlockSpec((_BR, d), lambda i: (i, 0)),
          pl.BlockSpec((1, D_OUT), lambda i: (0, 0)),
      ],
      out_specs=pl.BlockSpec((_BR, D_OUT), lambda i: (i, 0)),
      out_shape=jax.ShapeDtypeStruct((N_NODES, D_OUT), jnp.float32),
  )(a, c, h2, b)


def kernel(x, edge_index, W1l, W1r, b1, W2l, W2r, b2):
  ei = edge_index.astype(jnp.int32)
  npad = E_PAD - N_EDGES
  pad_i = jnp.arange(npad, dtype=jnp.int32)
  pad_src = pad_i % N_NODES
  pad_dst = N_NODES + pad_i % (N_PAD - N_NODES)
  edges = jnp.concatenate(
      [ei, jnp.stack([pad_src, pad_dst])], axis=1
  ).reshape(2, E_PAD // CHUNK, CHUNK)

  wcat1 = jnp.concatenate([W1l.T, W1r.T], axis=1)
  p1, xr = _proj1(x, wcat1)

  agg1, cnt = _agg_l1(p1.reshape(2 * N_PAD, D_HALF), edges)

  wcat2 = jnp.concatenate([W2l.T, W2r.T], axis=1)
  out2 = _layer_mid(agg1, cnt, xr, b1.reshape(1, -1), wcat2)

  agg2 = _agg_l2(out2.reshape(2 * N_PAD, D_HALF), edges)

  return _layer_out(agg2, cnt, out2, b2.reshape(1, -1))


# speedup vs baseline: 15.5131x; 1.0001x over previous
import functools

import jax
import jax.numpy as jnp
from jax import lax
from jax.experimental import pallas as pl
from jax.experimental.pallas import tpu as pltpu
from jax.experimental.pallas import tpu_sc as plsc

N_NODES = 10000
N_EDGES = 160000
D_IN = 256
D_HID = 128
D_OUT = 64

NC = 2
NS = 16
NW = NC * NS

N_PAD = 10240
E_PAD = 163840
CHUNK = 128
ROWS_PER_TILE = N_PAD // NS
CNT_W = 8
D_HALF = 64


def _make_sc_agg(n_chunks, feature_split, with_cnt, NBUF):
  D = D_HALF
  mesh = plsc.VectorSubcoreMesh(core_axis_name="c", subcore_axis_name="s")
  out_type = [jax.ShapeDtypeStruct((N_PAD, 2 * D), jnp.float32)]
  if with_cnt:
    out_type.append(jax.ShapeDtypeStruct((N_PAD, 2 * D), jnp.float32))
  scratch = [
      pltpu.VMEM((n_chunks, CHUNK), jnp.int32),
      pltpu.VMEM((n_chunks, CHUNK), jnp.int32),
      [pltpu.VMEM((CHUNK, D), jnp.float32) for _ in range(NBUF)],
      pltpu.VMEM_SHARED((N_PAD, D), jnp.float32),
      [pltpu.SemaphoreType.DMA for _ in range(NBUF)],
      [pltpu.SemaphoreType.DMA for _ in range(NBUF)],
  ]
  if with_cnt:
    scratch += [
        pltpu.VMEM((CHUNK, CNT_W), jnp.float32),
        pltpu.VMEM_SHARED((N_PAD, CNT_W), jnp.float32),
        [pltpu.SemaphoreType.DMA for _ in range(NBUF)],
    ]

  def body(p_hbm, edges_hbm, *rest):
    if with_cnt:
      (out_hbm, cnt_hbm, src_v, dst_v, rows, acc_sh, sem_g, sem_s, ones_v,
       cnt_sh, sem_c) = rest
    else:
      out_hbm, src_v, dst_v, rows, acc_sh, sem_g, sem_s = rest
    cid = lax.axis_index("c")
    sid = lax.axis_index("s")
    row0 = sid * ROWS_PER_TILE
    if feature_split:
      chunk0 = sid * n_chunks
    else:
      chunk0 = (sid * NC + cid) * n_chunks

    pltpu.sync_copy(edges_hbm.at[0, pl.ds(chunk0, n_chunks)], src_v)
    pltpu.sync_copy(edges_hbm.at[1, pl.ds(chunk0, n_chunks)], dst_v)
    half = cid if feature_split else 0

    def adjust(i, carry):
      for j in range(CHUNK // 16):
        sl = src_v[i, pl.ds(j * 16, 16)]
        src_v[i, pl.ds(j * 16, 16)] = sl + sl + half
      return carry
    lax.fori_loop(0, n_chunks, adjust, 0)

    def zero_row(i, carry):
      for j in range(D // 16):
        rows[0][i, pl.ds(j * 16, 16)] = jnp.zeros((16,), jnp.float32)
      return carry
    lax.fori_loop(0, CHUNK, zero_row, 0)
    for r in range(ROWS_PER_TILE // CHUNK):
      pltpu.sync_copy(rows[0], acc_sh.at[pl.ds(row0 + r * CHUNK, CHUNK)])
    if with_cnt:
      def zero_ones(i, carry):
        ones_v[i, :] = jnp.zeros((CNT_W,), jnp.float32)
        return carry
      lax.fori_loop(0, CHUNK, zero_ones, 0)
      for r in range(ROWS_PER_TILE // CHUNK):
        pltpu.sync_copy(ones_v, cnt_sh.at[pl.ds(row0 + r * CHUNK, CHUNK)])
      def fill_ones(i, carry):
        ones_v[i, :] = jnp.ones((CNT_W,), jnp.float32)
        return carry
      lax.fori_loop(0, CHUNK, fill_ones, 0)
    plsc.subcore_barrier()

    def gather(k, b):
      pltpu.async_copy(p_hbm.at[src_v.at[k]], rows[b], sem_g[b])

    def scatter(k, b):
      pltpu.async_copy(rows[b], acc_sh.at[dst_v.at[k]], sem_s[b], add=True)

    for b in range(NBUF - 1):
      gather(b, b)

    def step(j, carry):
      for b in range(NBUF):
        k = j * NBUF + b
        pltpu.make_async_copy(p_hbm.at[src_v.at[k]], rows[b],
                              sem_g[b]).wait()
        scatter(k, b)
        if with_cnt:
          @pl.when(k >= NBUF)
          def _wait_cnt(b=b, k=k):
            pltpu.make_async_copy(ones_v, cnt_sh.at[dst_v.at[k - NBUF]],
                                  sem_c[b]).wait()
          pltpu.async_copy(ones_v, cnt_sh.at[dst_v.at[k]], sem_c[b], add=True)
        bn = (b + NBUF - 1) % NBUF
        kn = k + NBUF - 1

        @pl.when(k >= 1)
        def _wait_prev(bn=bn, kn=kn):
          pltpu.make_async_copy(rows[bn], acc_sh.at[dst_v.at[kn - NBUF]],
                                sem_s[bn]).wait()

        @pl.when(kn < n_chunks)
        def _prefetch(bn=bn, kn=kn):
          gather(kn, bn)
      return carry
    lax.fori_loop(0, n_chunks // NBUF, step, 0)
    bl = (n_chunks - 1) % NBUF
    pltpu.make_async_copy(rows[bl], acc_sh.at[dst_v.at[n_chunks - 1]],
                          sem_s[bl]).wait()
    if with_cnt:
      for b in range(NBUF):
        kf = n_chunks - NBUF + b
        pltpu.make_async_copy(ones_v, cnt_sh.at[dst_v.at[kf]],
                              sem_c[b]).wait()

    plsc.subcore_barrier()
    col0 = cid * D
    pltpu.sync_copy(acc_sh.at[pl.ds(row0, ROWS_PER_TILE)],
                    out_hbm.at[pl.ds(row0, ROWS_PER_TILE), pl.ds(col0, D)])
    if with_cnt:
      @pl.when(cid == 0)
      def _cnt_out():
        pltpu.sync_copy(cnt_sh.at[pl.ds(row0, ROWS_PER_TILE)],
                        cnt_hbm.at[pl.ds(row0, ROWS_PER_TILE),
                                   pl.ds(0, CNT_W)])

  return pl.kernel(body, out_type=out_type, mesh=mesh, scratch_types=scratch,
                   compiler_params=pltpu.CompilerParams(
                       use_tc_tiling_on_sc=False))


_make_sc_agg = functools.lru_cache(maxsize=None)(_make_sc_agg)


def _agg_l1(p_tab, edges):
  return _make_sc_agg(E_PAD // CHUNK // NS, True, True, 5)(p_tab, edges)


def _agg_l2(p_tab, edges):
  out = _make_sc_agg(E_PAD // CHUNK // NW, False, False, 8)(p_tab, edges)
  if isinstance(out, (list, tuple)):
    out = out[0]
  return out


_BR = 2048


def _proj1_body(x_ref, w_ref, p_ref, xr_ref):
  w = w_ref[...].astype(jnp.bfloat16)
  x = x_ref[...].astype(jnp.bfloat16)
  p_ref[...] = jnp.dot(x, w[:, :D_HID], preferred_element_type=jnp.float32)
  xr_ref[...] = jnp.dot(x, w[:, D_HID:], preferred_element_type=jnp.float32)


def _proj1(x, w):
  k = x.shape[1]
  n = w.shape[1]
  return pl.pallas_call(
      _proj1_body,
      grid=(N_PAD // _BR,),
      in_specs=[
          pl.BlockSpec((_BR, k), lambda i: (i, 0)),
          pl.BlockSpec((k, n), lambda i: (0, 0)),
      ],
      out_specs=[
          pl.BlockSpec((_BR, D_HID), lambda i: (i, 0)),
          pl.BlockSpec((_BR, D_HID), lambda i: (i, 0)),
      ],
      out_shape=[
          jax.ShapeDtypeStruct((N_PAD, D_HID), jnp.float32),
          jax.ShapeDtypeStruct((N_PAD, D_HID), jnp.float32),
      ],
  )(x, w)


def _mid_body(s_ref, c_ref, xr_ref, b_ref, w_ref, o_ref):
  rcnt = 1.0 / jnp.maximum(c_ref[:, 0:1], 1.0)
  h = s_ref[...] * rcnt + xr_ref[...] + b_ref[...]
  h = jnp.maximum(h, 0.0).astype(jnp.bfloat16)
  o_ref[...] = jnp.dot(h, w_ref[...].astype(jnp.bfloat16),
                       preferred_element_type=jnp.float32)


def _layer_mid(s, c, xr, b, w):
  m, d = s.shape
  n = w.shape[1]
  return pl.pallas_call(
      _mid_body,
      grid=(m // _BR,),
      in_specs=[
          pl.BlockSpec((_BR, d), lambda i: (i, 0)),
          pl.BlockSpec((_BR, d), lambda i: (i, 0)),
          pl.BlockSpec((_BR, d), lambda i: (i, 0)),
          pl.BlockSpec((1, d), lambda i: (0, 0)),
          pl.BlockSpec((d, n), lambda i: (0, 0)),
      ],
      out_specs=pl.BlockSpec((_BR, n), lambda i: (i, 0)),
      out_shape=jax.ShapeDtypeStruct((m, n), jnp.float32),
  )(s, c, xr, b, w)


def _out_body(a_ref, c_ref, h2_ref, b_ref, o_ref):
  rcnt = 1.0 / jnp.maximum(c_ref[:, 0:1], 1.0)
  s2 = a_ref[:, :D_OUT] + a_ref[:, D_OUT:]
  o = s2 * rcnt + h2_ref[:, D_OUT:] + b_ref[...]
  m = jnp.max(o, axis=-1, keepdims=True)
  e = jnp.exp(o - m)
  lse = m + jnp.log(jnp.sum(e, axis=-1, keepdims=True))
  o_ref[...] = o - lse


def _layer_out(a, c, h2, b):
  m, d = a.shape
  return pl.pallas_call(
      _out_body,
      grid=(m // _BR,),
      in_specs=[
          pl.BlockSpec((_BR, d), lambda i: (i, 0)),
          pl.BlockSpec((_BR, d), lambda i: (i, 0)),
          pl.BlockSpec((_BR, d), lambda i: (i, 0)),
          pl.BlockSpec((1, D_OUT), lambda i: (0, 0)),
      ],
      out_specs=pl.BlockSpec((_BR, D_OUT), lambda i: (i, 0)),
      out_shape=jax.ShapeDtypeStruct((N_NODES, D_OUT), jnp.float32),
  )(a, c, h2, b)


def kernel(x, edge_index, W1l, W1r, b1, W2l, W2r, b2):
  ei = edge_index.astype(jnp.int32)
  npad = E_PAD - N_EDGES
  pad_i = jnp.arange(npad, dtype=jnp.int32)
  pad_src = pad_i % N_NODES
  pad_dst = N_NODES + pad_i % (N_PAD - N_NODES)
  edges = jnp.concatenate(
      [ei, jnp.stack([pad_src, pad_dst])], axis=1
  ).reshape(2, E_PAD // CHUNK, CHUNK)

  wcat1 = jnp.concatenate([W1l.T, W1r.T], axis=1)
  p1, xr = _proj1(x, wcat1)

  agg1, cnt = _agg_l1(p1.reshape(2 * N_PAD, D_HALF), edges)

  wcat2 = jnp.concatenate([W2l.T, W2r.T], axis=1)
  out2 = _layer_mid(agg1, cnt, xr, b1.reshape(1, -1), wcat2)

  agg2 = _agg_l2(out2.reshape(2 * N_PAD, D_HALF), edges)

  return _layer_out(agg2, cnt, out2, b2.reshape(1, -1))
